# baseline (speedup 1.0000x reference)
import dataclasses
import functools

import jax
import jax.numpy as jnp
from jax import lax
from jax.experimental import pallas as pl
from jax.experimental.pallas import tpu as pltpu
from jax.experimental.pallas import tpu_sc as plsc

E = 3200000
G = 512
A = 2048
V = 10000
VP = 10112
PT = 0.5
EPS = 0.05
ALPHA = 1.0
BASE = 10.0
TBONUS = 1.0
ILLEGAL = 1e-08
LP = 0.9
PF1P = 1.0


def _final_math_kernel(stats_ref, hit_ref, ag_ref, ptr_lo_ref, ptr_hi_ref,
                       reached_ref, rs_ref, rf_ref,
                       reward_o, recall_o, success_o, zeros_o, fallback_o,
                       pos_p_o, pos_r_o, pos_f1_o, ans_p_o, ans_r_o, ans_f1_o,
                       path_p_o, path_r_o, path_f1_o, has_gt_o, pfh_o,
                       rpt_o, sem_o):
    selected_total = stats_ref[0:1, :]
    pos_total = stats_ref[1:2, :]
    selected_pos = stats_ref[2:3, :]
    path_total = stats_ref[3:4, :]
    path_hits = stats_ref[4:5, :]
    sel_scores_sum = stats_ref[5:6, :]

    def prf(hits, pred, tgt):
        z = jnp.zeros_like(hits)
        p = jnp.where(pred > 0, hits / jnp.clip(pred, 1.0), z)
        r = jnp.where(tgt > 0, hits / jnp.clip(tgt, 1.0), z)
        f1 = jnp.where(p + r > 0, 2.0 * p * r / jnp.clip(p + r, 1e-08), z)
        return p, r, f1

    fallback = (selected_total == 0).astype(jnp.float32)
    pos_p, pos_r, pos_f1 = prf(selected_pos, selected_total, pos_total)
    label_recall = jnp.where(pos_total > 0,
                             selected_pos / jnp.clip(pos_total, 1.0),
                             jnp.zeros_like(selected_pos))

    onehot = (ag_ref[...] == lax.broadcasted_iota(jnp.int32, (A, G), 1))
    hits = jnp.dot(hit_ref[...].astype(jnp.bfloat16),
                   onehot.astype(jnp.bfloat16),
                   preferred_element_type=jnp.float32)
    ans_counts = (ptr_hi_ref[...] - ptr_lo_ref[...]).astype(jnp.float32)
    reached_total = reached_ref[...]
    ans_p, ans_r, ans_f1 = prf(hits, reached_total, ans_counts)
    has_answers = ans_counts > 0
    recall = jnp.where(has_answers, ans_r, label_recall)

    path_p, path_r, path_f1 = prf(path_hits, selected_total, path_total)
    has_gt_path = path_total > 0
    path_full_hit = jnp.logical_and(has_gt_path, path_hits >= path_total)
    recall = jnp.where(has_gt_path, path_r, recall)
    success = rs_ref[...] > 0.5
    any_path = jnp.any(has_gt_path)
    success = jnp.logical_and(
        success, jnp.logical_or(jnp.logical_not(any_path), path_full_hit))
    rf = rf_ref[...]
    connectivity = jnp.clip(rf + EPS, 1e-06)
    semantic_mean = jnp.clip(
        jnp.where(selected_total > 0,
                  sel_scores_sum / jnp.clip(selected_total, 1.0),
                  jnp.zeros_like(sel_scores_sum)), 1e-08, 1.0)
    semantic_score = semantic_mean ** ALPHA
    reward_path_term = jnp.full_like(selected_total, LP) ** selected_total
    path_term = jnp.ones_like(reward_path_term)
    path_term = jnp.where(jnp.logical_and(any_path, has_gt_path),
                          jnp.clip(path_f1, 0.001) ** PF1P, path_term)
    reward = jnp.where(success,
                       BASE * reward_path_term * path_term * semantic_score
                       * connectivity,
                       jnp.full_like(connectivity, ILLEGAL))
    reward = jnp.where(jnp.logical_and(any_path, path_full_hit),
                       reward * (1.0 + TBONUS), reward)
    reward = jnp.clip(reward, ILLEGAL)

    reward_o[...] = reward
    recall_o[...] = recall
    success_o[...] = success.astype(jnp.float32)
    zeros_o[...] = jnp.zeros_like(recall)
    fallback_o[...] = fallback
    pos_p_o[...] = pos_p
    pos_r_o[...] = pos_r
    pos_f1_o[...] = pos_f1
    ans_p_o[...] = ans_p
    ans_r_o[...] = ans_r
    ans_f1_o[...] = ans_f1
    path_p_o[...] = path_p
    path_r_o[...] = path_r
    path_f1_o[...] = path_f1
    has_gt_o[...] = has_gt_path.astype(jnp.float32)
    pfh_o[...] = path_full_hit.astype(jnp.float32)
    rpt_o[...] = reward_path_term
    sem_o[...] = semantic_score


def _final_math(stats, answer_hit, ans_graph2d, ptr_lo, ptr_hi, reached_total,
                rs_f, rf):
    o = jax.ShapeDtypeStruct((1, G), jnp.float32)
    return pl.pallas_call(
        _final_math_kernel,
        out_shape=[o] * 18,
    )(stats, answer_hit, ans_graph2d, ptr_lo, ptr_hi, reached_total, rs_f, rf)


NC = 2
NS = 16
NW = NC * NS
GPW = G // NW
BLK = 4096
NROW = BLK // 128
T = (G + 1) * VP
DUMMY = G * VP


def _sc_compiler_params():
    cp = pltpu.CompilerParams()
    if "needs_layout_passes" in pltpu.CompilerParams.__dataclass_fields__:
        cp = dataclasses.replace(cp, needs_layout_passes=False)
    return cp


def _sc_main(eb, tl, hd, sel_f, lab, sco, pth, bounds):
    mesh = plsc.VectorSubcoreMesh(core_axis_name="c", subcore_axis_name="s")

    @functools.partial(
        pl.kernel,
        out_type=[jax.ShapeDtypeStruct((NW, 8, 16), jnp.float32),
                  jax.ShapeDtypeStruct((T,), jnp.float32)],
        mesh=mesh,
        scratch_types=[
            pltpu.VMEM((16,), jnp.int32),
            pltpu.VMEM((BLK,), jnp.int32),
            pltpu.VMEM((BLK,), jnp.int32),
            pltpu.VMEM((BLK,), jnp.int32),
            pltpu.VMEM((BLK,), jnp.float32),
            pltpu.VMEM((BLK,), jnp.float32),
            pltpu.VMEM((BLK,), jnp.float32),
            pltpu.VMEM((BLK,), jnp.float32),
            pltpu.VMEM((96, 16), jnp.float32),
            pltpu.VMEM((NROW, 128), jnp.int32),
            pltpu.VMEM((NROW, 128), jnp.int32),
            pltpu.VMEM((128,), jnp.float32),
            pltpu.VMEM((VP,), jnp.float32),
            pltpu.VMEM((8, 16), jnp.float32),
            pltpu.SemaphoreType.DMA,
            pltpu.SemaphoreType.DMA,
            pltpu.SemaphoreType.DMA,
            pltpu.SemaphoreType.DMA,
        ],
        compiler_params=_sc_compiler_params())
    def k(eb_h, tl_h, hd_h, sel_h, lab_h, sco_h, pth_h, bnd_h,
          seg_o, table_o,
          bnd_v, b_eb, b_tl, b_hd, b_sel, b_lab, b_sco, b_pth,
          acc, idx_t, idx_h, ones_v, zrow, stage,
          sem_in, sem_z, sem_sc, sem_out):
        wid = lax.axis_index("s") * NC + lax.axis_index("c")
        g0 = wid * GPW
        lane = lax.iota(jnp.int32, 16)
        zero16 = jnp.zeros((16,), jnp.float32)
        one16 = jnp.full((16,), 1.0, jnp.float32)

        pltpu.async_copy(bnd_h.at[wid], bnd_v, sem_in).wait()
        bv = bnd_v[...]
        ws8 = jnp.max(jnp.where(lane == 0, bv, 0))
        ws = jnp.max(jnp.where(lane == 1, bv, 0))
        we = jnp.max(jnp.where(lane == 2, bv, 0))
        nst = jnp.max(jnp.where(lane == 3, bv, 0))

        @pl.loop(0, VP // 16)
        def _(i):
            zrow[pl.ds(i * 16, 16)] = zero16

        @pl.loop(0, 8)
        def _(i):
            ones_v[pl.ds(i * 16, 16)] = one16

        @pl.loop(0, 96)
        def _(r):
            acc[r, :] = zero16

        @pl.loop(0, GPW)
        def _(j):
            pltpu.async_copy(zrow, table_o.at[pl.ds(pl.multiple_of((g0 + j) * VP, 8), VP)], sem_z)

        @pl.loop(0, GPW)
        def _(j):
            pltpu.make_async_copy(
                zrow, table_o.at[pl.ds(pl.multiple_of((g0 + j) * VP, 8), VP)], sem_z).wait()

        def block(i, carry):
            off = pl.multiple_of(jnp.minimum(ws8 + i * BLK, E - BLK), 8)
            cps = [pltpu.async_copy(eb_h.at[pl.ds(off, BLK)], b_eb, sem_in),
                   pltpu.async_copy(tl_h.at[pl.ds(off, BLK)], b_tl, sem_in),
                   pltpu.async_copy(hd_h.at[pl.ds(off, BLK)], b_hd, sem_in),
                   pltpu.async_copy(sel_h.at[pl.ds(off, BLK)], b_sel, sem_in),
                   pltpu.async_copy(lab_h.at[pl.ds(off, BLK)], b_lab, sem_in),
                   pltpu.async_copy(sco_h.at[pl.ds(off, BLK)], b_sco, sem_in),
                   pltpu.async_copy(pth_h.at[pl.ds(off, BLK)], b_pth, sem_in)]
            for c in cps:
                c.wait()

            @pl.loop(0, NROW)
            def _(j):
                for m in range(8):
                    base = j * 128 + m * 16
                    eb_v = b_eb[pl.ds(base, 16)]
                    gi = off + base + lane
                    m_in = jnp.logical_and(gi >= ws, gi < we)
                    selb = b_sel[pl.ds(base, 16)] > 0.0
                    msel = jnp.logical_and(m_in, selb)
                    tb = eb_v * VP
                    it = jnp.where(msel, tb + b_tl[pl.ds(base, 16)], DUMMY)
                    ih = jnp.where(msel, tb + b_hd[pl.ds(base, 16)], DUMMY)
                    idx_t[j, pl.ds(m * 16, 16)] = it
                    idx_h[j, pl.ds(m * 16, 16)] = ih
                    rowc = jnp.clip(eb_v - g0, 0, GPW - 1)
                    pos_v = (b_lab[pl.ds(base, 16)] > PT).astype(jnp.float32)
                    eff_v = jnp.clip(b_sco[pl.ds(base, 16)], 1e-08, 1.0)
                    pth_v = b_pth[pl.ds(base, 16)]
                    plsc.addupdate_scatter(acc, [rowc, lane], one16, mask=msel)
                    plsc.addupdate_scatter(acc, [rowc + 16, lane], pos_v,
                                           mask=m_in)
                    plsc.addupdate_scatter(acc, [rowc + 32, lane], pos_v,
                                           mask=msel)
                    plsc.addupdate_scatter(acc, [rowc + 48, lane], pth_v,
                                           mask=m_in)
                    plsc.addupdate_scatter(acc, [rowc + 64, lane], pth_v,
                                           mask=msel)
                    plsc.addupdate_scatter(acc, [rowc + 80, lane], eff_v,
                                           mask=msel)
                pltpu.async_copy(ones_v, table_o.at[idx_t.at[j]], sem_sc)
                pltpu.async_copy(ones_v, table_o.at[idx_h.at[j]], sem_sc)

            @pl.loop(0, NROW)
            def _(j):
                pltpu.make_async_copy(
                    ones_v, table_o.at[idx_t.at[j]], sem_sc).wait()
                pltpu.make_async_copy(
                    ones_v, table_o.at[idx_h.at[j]], sem_sc).wait()
            return carry

        lax.fori_loop(0, nst, block, 0)

        for s in range(6):
            row_s = zero16
            for j in range(GPW):
                row_s = jnp.where(lane == j, jnp.sum(acc[s * GPW + j, :]),
                                  row_s)
            stage[s, :] = row_s

        pltpu.async_copy(stage, seg_o.at[wid], sem_out).wait()

    return k(eb, tl, hd, sel_f, lab, sco, pth, bounds)


def _sc_gather(table, aidx):
    mesh = plsc.VectorSubcoreMesh(core_axis_name="c", subcore_axis_name="s")
    apw = A // NW

    @functools.partial(
        pl.kernel,
        out_type=jax.ShapeDtypeStruct((A,), jnp.float32),
        mesh=mesh,
        scratch_types=[pltpu.VMEM((apw,), jnp.int32),
                       pltpu.VMEM((apw,), jnp.float32),
                       pltpu.SemaphoreType.DMA],
        compiler_params=_sc_compiler_params())
    def k(tab_h, ai_h, out_h, ai_v, hit_v, sem):
        wid = lax.axis_index("s") * NC + lax.axis_index("c")
        base = wid * apw
        pltpu.async_copy(ai_h.at[pl.ds(base, apw)], ai_v, sem).wait()
        pltpu.async_copy(tab_h.at[ai_v], hit_v, sem).wait()
        pltpu.async_copy(hit_v, out_h.at[pl.ds(base, apw)], sem).wait()

    return k(table, aidx)


def _rowsum_kernel(t_ref, o_ref):
    o_ref[...] = jnp.sum(t_ref[...], axis=1, keepdims=True)


def _table_rowsum(table2d):
    return pl.pallas_call(
        _rowsum_kernel,
        grid=(G // 8,),
        in_specs=[pl.BlockSpec((8, VP), lambda i: (i, 0))],
        out_specs=pl.BlockSpec((8, 1), lambda i: (i, 0)),
        out_shape=jax.ShapeDtypeStruct((G, 1), jnp.float32),
    )(table2d)


def kernel(selected_mask, edge_labels, edge_scores, edge_batch, edge_heads,
           edge_tails, answer_entity_ids, answer_ptr, path_mask, path_exists,
           reach_success, reach_fraction):
    sel_f = selected_mask.astype(jnp.float32)
    path_f = path_mask.astype(jnp.float32)
    eb = edge_batch.astype(jnp.int32)

    start = jnp.searchsorted(eb, jnp.arange(G + 1), side="left"
                             ).astype(jnp.int32)
    ws = start[0:G:GPW]
    we = start[GPW:G + 1:GPW]
    ws8 = (ws // 8) * 8
    nst = (we - ws8 + BLK - 1) // BLK
    bounds = (jnp.zeros((NW, 16), jnp.int32)
              .at[:, 0].set(ws8).at[:, 1].set(ws)
              .at[:, 2].set(we).at[:, 3].set(nst))

    seg_out, table = _sc_main(eb, edge_tails.astype(jnp.int32),
                              edge_heads.astype(jnp.int32), sel_f,
                              edge_labels, edge_scores, path_f, bounds)
    stats = jnp.transpose(seg_out, (1, 0, 2)).reshape(8, G)

    ans_graph = jnp.clip(
        jnp.searchsorted(answer_ptr, jnp.arange(A), side="right") - 1, 0, G - 1
    ).astype(jnp.int32)
    aidx = ans_graph * VP + answer_entity_ids.astype(jnp.int32)
    answer_hit = _sc_gather(table, aidx).reshape(1, A)
    table2d = table[:G * VP].reshape(G, VP)

    reached_total = _table_rowsum(table2d).reshape(1, G)
    outs = _final_math(
        stats,
        answer_hit,
        ans_graph.reshape(A, 1),
        answer_ptr[:-1].reshape(1, G).astype(jnp.int32),
        answer_ptr[1:].reshape(1, G).astype(jnp.int32),
        reached_total,
        reach_success.astype(jnp.float32).reshape(1, G),
        reach_fraction.astype(jnp.float32).reshape(1, G),
    )
    (reward, recall, success_f, zeros, fallback, pos_p, pos_r, pos_f1, ans_p,
     ans_r, ans_f1, path_p, path_r, path_f1, has_gt_f, pfh_f, rpt,
     sem) = [o.reshape(G) for o in outs]
    rf = reach_fraction.astype(jnp.float32)
    return (reward, recall, success_f, zeros, fallback, pos_p, pos_r, pos_f1,
            ans_p, ans_r, ans_f1, path_p, path_r, path_f1,
            has_gt_f.astype(bool), pfh_f, rf, path_exists, rf, rpt, sem)


# speedup vs baseline: 845.4136x; 845.4136x over previous
import dataclasses
import functools

import jax
import jax.numpy as jnp
from jax import lax
from jax.experimental import pallas as pl
from jax.experimental.pallas import tpu as pltpu
from jax.experimental.pallas import tpu_sc as plsc

E = 3200000
G = 512
A = 2048
V = 10000
VP = 10112
PT = 0.5
EPS = 0.05
ALPHA = 1.0
BASE = 10.0
TBONUS = 1.0
ILLEGAL = 1e-08
LP = 0.9
PF1P = 1.0


def _final_math_kernel(stats_ref, hit_ref, ag_ref, ptr_lo_ref, ptr_hi_ref,
                       reached_ref, rs_ref, rf_ref,
                       reward_o, recall_o, success_o, zeros_o, fallback_o,
                       pos_p_o, pos_r_o, pos_f1_o, ans_p_o, ans_r_o, ans_f1_o,
                       path_p_o, path_r_o, path_f1_o, has_gt_o, pfh_o,
                       rpt_o, sem_o):
    selected_total = stats_ref[0:1, :]
    pos_total = stats_ref[1:2, :]
    selected_pos = stats_ref[2:3, :]
    path_total = stats_ref[3:4, :]
    path_hits = stats_ref[4:5, :]
    sel_scores_sum = stats_ref[5:6, :]

    def prf(hits, pred, tgt):
        z = jnp.zeros_like(hits)
        p = jnp.where(pred > 0, hits / jnp.clip(pred, 1.0), z)
        r = jnp.where(tgt > 0, hits / jnp.clip(tgt, 1.0), z)
        f1 = jnp.where(p + r > 0, 2.0 * p * r / jnp.clip(p + r, 1e-08), z)
        return p, r, f1

    fallback = (selected_total == 0).astype(jnp.float32)
    pos_p, pos_r, pos_f1 = prf(selected_pos, selected_total, pos_total)
    label_recall = jnp.where(pos_total > 0,
                             selected_pos / jnp.clip(pos_total, 1.0),
                             jnp.zeros_like(selected_pos))

    onehot = (ag_ref[...] == lax.broadcasted_iota(jnp.int32, (A, G), 1))
    hits = jnp.dot(hit_ref[...].astype(jnp.bfloat16),
                   onehot.astype(jnp.bfloat16),
                   preferred_element_type=jnp.float32)
    ans_counts = (ptr_hi_ref[...] - ptr_lo_ref[...]).astype(jnp.float32)
    reached_total = reached_ref[...]
    ans_p, ans_r, ans_f1 = prf(hits, reached_total, ans_counts)
    has_answers = ans_counts > 0
    recall = jnp.where(has_answers, ans_r, label_recall)

    path_p, path_r, path_f1 = prf(path_hits, selected_total, path_total)
    has_gt_path = path_total > 0
    path_full_hit = jnp.logical_and(has_gt_path, path_hits >= path_total)
    recall = jnp.where(has_gt_path, path_r, recall)
    success = rs_ref[...] > 0.5
    any_path = jnp.any(has_gt_path)
    success = jnp.logical_and(
        success, jnp.logical_or(jnp.logical_not(any_path), path_full_hit))
    rf = rf_ref[...]
    connectivity = jnp.clip(rf + EPS, 1e-06)
    semantic_mean = jnp.clip(
        jnp.where(selected_total > 0,
                  sel_scores_sum / jnp.clip(selected_total, 1.0),
                  jnp.zeros_like(sel_scores_sum)), 1e-08, 1.0)
    semantic_score = semantic_mean ** ALPHA
    reward_path_term = jnp.full_like(selected_total, LP) ** selected_total
    path_term = jnp.ones_like(reward_path_term)
    path_term = jnp.where(jnp.logical_and(any_path, has_gt_path),
                          jnp.clip(path_f1, 0.001) ** PF1P, path_term)
    reward = jnp.where(success,
                       BASE * reward_path_term * path_term * semantic_score
                       * connectivity,
                       jnp.full_like(connectivity, ILLEGAL))
    reward = jnp.where(jnp.logical_and(any_path, path_full_hit),
                       reward * (1.0 + TBONUS), reward)
    reward = jnp.clip(reward, ILLEGAL)

    reward_o[...] = reward
    recall_o[...] = recall
    success_o[...] = success.astype(jnp.float32)
    zeros_o[...] = jnp.zeros_like(recall)
    fallback_o[...] = fallback
    pos_p_o[...] = pos_p
    pos_r_o[...] = pos_r
    pos_f1_o[...] = pos_f1
    ans_p_o[...] = ans_p
    ans_r_o[...] = ans_r
    ans_f1_o[...] = ans_f1
    path_p_o[...] = path_p
    path_r_o[...] = path_r
    path_f1_o[...] = path_f1
    has_gt_o[...] = has_gt_path.astype(jnp.float32)
    pfh_o[...] = path_full_hit.astype(jnp.float32)
    rpt_o[...] = reward_path_term
    sem_o[...] = semantic_score


def _final_math(stats, answer_hit, ans_graph2d, ptr_lo, ptr_hi, reached_total,
                rs_f, rf):
    o = jax.ShapeDtypeStruct((1, G), jnp.float32)
    return pl.pallas_call(
        _final_math_kernel,
        out_shape=[o] * 18,
    )(stats, answer_hit, ans_graph2d, ptr_lo, ptr_hi, reached_total, rs_f, rf)


NC = 2
NS = 16
NW = NC * NS
GPW = G // NW
BLK = 4096
NROW = BLK // 128
T = G * VP


def _sc_compiler_params():
    cp = pltpu.CompilerParams()
    if "needs_layout_passes" in pltpu.CompilerParams.__dataclass_fields__:
        cp = dataclasses.replace(cp, needs_layout_passes=False)
    return cp


def _sc_main(tl, hd, sel_f, lab, sco, pth, start520):
    mesh = plsc.VectorSubcoreMesh(core_axis_name="c", subcore_axis_name="s")

    @functools.partial(
        pl.kernel,
        out_type=[jax.ShapeDtypeStruct((NW, 8, 16), jnp.float32),
                  jax.ShapeDtypeStruct((T,), jnp.float32)],
        mesh=mesh,
        scratch_types=[
            pltpu.VMEM((520,), jnp.int32),
            pltpu.VMEM((BLK,), jnp.int32),
            pltpu.VMEM((BLK,), jnp.int32),
            pltpu.VMEM((BLK,), jnp.float32),
            pltpu.VMEM((BLK,), jnp.float32),
            pltpu.VMEM((BLK,), jnp.float32),
            pltpu.VMEM((BLK,), jnp.float32),
            pltpu.VMEM((VP,), jnp.float32),
            pltpu.VMEM((8, 16), jnp.float32),
            pltpu.SemaphoreType.DMA,
            pltpu.SemaphoreType.DMA,
            pltpu.SemaphoreType.DMA,
        ],
        compiler_params=_sc_compiler_params())
    def k(tl_h, hd_h, sel_h, lab_h, sco_h, pth_h, st_h,
          seg_o, table_o,
          startv, b_tl, b_hd, b_sel, b_lab, b_sco, b_pth, rowbuf, stage,
          sem_in, sem_t, sem_out):
        wid = lax.axis_index("s") * NC + lax.axis_index("c")
        g0 = wid * GPW
        lane = lax.iota(jnp.int32, 16)
        zero16 = jnp.zeros((16,), jnp.float32)
        one16 = jnp.full((16,), 1.0, jnp.float32)

        pltpu.async_copy(st_h, startv, sem_in).wait()

        def graph_body(j, rows):
            g = g0 + j
            iv = jnp.where(lane < 8, g, g + 1)
            sv = plsc.load_gather(startv, [iv])
            es = jnp.max(jnp.where(lane < 8, sv, 0))
            ee = jnp.max(jnp.where(lane >= 8, sv, 0))
            es8 = (es // 8) * 8
            nst = (ee - es8 + BLK - 1) // BLK

            @pl.loop(0, VP // 16)
            def _(i):
                rowbuf[pl.ds(i * 16, 16)] = zero16

            def block(i, sums):
                off = pl.multiple_of(jnp.minimum(es8 + i * BLK, E - BLK), 8)
                cps = [
                    pltpu.async_copy(tl_h.at[pl.ds(off, BLK)], b_tl, sem_in),
                    pltpu.async_copy(hd_h.at[pl.ds(off, BLK)], b_hd, sem_in),
                    pltpu.async_copy(sel_h.at[pl.ds(off, BLK)], b_sel, sem_in),
                    pltpu.async_copy(lab_h.at[pl.ds(off, BLK)], b_lab, sem_in),
                    pltpu.async_copy(sco_h.at[pl.ds(off, BLK)], b_sco, sem_in),
                    pltpu.async_copy(pth_h.at[pl.ds(off, BLK)], b_pth, sem_in),
                ]
                for c in cps:
                    c.wait()

                def row(j2, sums):
                    s0, s1, s2, s3, s4, s5 = sums
                    for m in range(8):
                        base = j2 * 128 + m * 16
                        gi = off + base + lane
                        m_in = jnp.logical_and(gi >= es, gi < ee)
                        selb = b_sel[pl.ds(base, 16)] > 0.0
                        msel = jnp.logical_and(m_in, selb)
                        plsc.store_scatter(rowbuf, [b_tl[pl.ds(base, 16)]],
                                           one16, mask=msel)
                        plsc.store_scatter(rowbuf, [b_hd[pl.ds(base, 16)]],
                                           one16, mask=msel)
                        pos_v = (b_lab[pl.ds(base, 16)] > PT
                                 ).astype(jnp.float32)
                        eff_v = jnp.clip(b_sco[pl.ds(base, 16)], 1e-08, 1.0)
                        pth_v = b_pth[pl.ds(base, 16)]
                        m_in_f = m_in.astype(jnp.float32)
                        msel_f = msel.astype(jnp.float32)
                        s0 = s0 + msel_f
                        s1 = s1 + m_in_f * pos_v
                        s2 = s2 + msel_f * pos_v
                        s3 = s3 + m_in_f * pth_v
                        s4 = s4 + msel_f * pth_v
                        s5 = s5 + msel_f * eff_v
                    return (s0, s1, s2, s3, s4, s5)

                return pl.loop(0, NROW, init_carry=sums)(row)

            sums = lax.fori_loop(0, nst, block, (zero16,) * 6)
            pltpu.async_copy(
                rowbuf,
                table_o.at[pl.ds(pl.multiple_of(g * VP, 8), VP)],
                sem_t).wait()
            return tuple(
                jnp.where(lane == j, jnp.sum(s), r)
                for s, r in zip(sums, rows))

        rows = pl.loop(0, GPW, init_carry=(zero16,) * 6)(graph_body)
        for s in range(6):
            stage[s, :] = rows[s]
        pltpu.async_copy(stage, seg_o.at[wid], sem_out).wait()

    return k(tl, hd, sel_f, lab, sco, pth, start520)


def _sc_gather(table, aidx):
    mesh = plsc.VectorSubcoreMesh(core_axis_name="c", subcore_axis_name="s")
    apw = A // NW

    @functools.partial(
        pl.kernel,
        out_type=jax.ShapeDtypeStruct((A,), jnp.float32),
        mesh=mesh,
        scratch_types=[pltpu.VMEM((apw,), jnp.int32),
                       pltpu.VMEM((apw,), jnp.float32),
                       pltpu.SemaphoreType.DMA],
        compiler_params=_sc_compiler_params())
    def k(tab_h, ai_h, out_h, ai_v, hit_v, sem):
        wid = lax.axis_index("s") * NC + lax.axis_index("c")
        base = wid * apw
        pltpu.async_copy(ai_h.at[pl.ds(base, apw)], ai_v, sem).wait()
        pltpu.async_copy(tab_h.at[ai_v], hit_v, sem).wait()
        pltpu.async_copy(hit_v, out_h.at[pl.ds(base, apw)], sem).wait()

    return k(table, aidx)


def _rowsum_kernel(t_ref, o_ref):
    o_ref[...] = jnp.sum(t_ref[...], axis=1, keepdims=True)


def _table_rowsum(table2d):
    return pl.pallas_call(
        _rowsum_kernel,
        grid=(G // 8,),
        in_specs=[pl.BlockSpec((8, VP), lambda i: (i, 0))],
        out_specs=pl.BlockSpec((8, 1), lambda i: (i, 0)),
        out_shape=jax.ShapeDtypeStruct((G, 1), jnp.float32),
    )(table2d)


def kernel(selected_mask, edge_labels, edge_scores, edge_batch, edge_heads,
           edge_tails, answer_entity_ids, answer_ptr, path_mask, path_exists,
           reach_success, reach_fraction):
    sel_f = selected_mask.astype(jnp.float32)
    path_f = path_mask.astype(jnp.float32)
    eb = edge_batch.astype(jnp.int32)

    start = jnp.searchsorted(eb, jnp.arange(G + 1), side="left"
                             ).astype(jnp.int32)
    start520 = jnp.pad(start, (0, 520 - (G + 1)))

    seg_out, table = _sc_main(edge_tails.astype(jnp.int32),
                              edge_heads.astype(jnp.int32), sel_f,
                              edge_labels, edge_scores, path_f, start520)
    stats = jnp.transpose(seg_out, (1, 0, 2)).reshape(8, G)

    ans_graph = jnp.clip(
        jnp.searchsorted(answer_ptr, jnp.arange(A), side="right") - 1, 0, G - 1
    ).astype(jnp.int32)
    aidx = ans_graph * VP + answer_entity_ids.astype(jnp.int32)
    answer_hit = _sc_gather(table, aidx).reshape(1, A)
    table2d = table.reshape(G, VP)

    reached_total = _table_rowsum(table2d).reshape(1, G)
    outs = _final_math(
        stats,
        answer_hit,
        ans_graph.reshape(A, 1),
        answer_ptr[:-1].reshape(1, G).astype(jnp.int32),
        answer_ptr[1:].reshape(1, G).astype(jnp.int32),
        reached_total,
        reach_success.astype(jnp.float32).reshape(1, G),
        reach_fraction.astype(jnp.float32).reshape(1, G),
    )
    (reward, recall, success_f, zeros, fallback, pos_p, pos_r, pos_f1, ans_p,
     ans_r, ans_f1, path_p, path_r, path_f1, has_gt_f, pfh_f, rpt,
     sem) = [o.reshape(G) for o in outs]
    rf = reach_fraction.astype(jnp.float32)
    return (reward, recall, success_f, zeros, fallback, pos_p, pos_r, pos_f1,
            ans_p, ans_r, ans_f1, path_p, path_r, path_f1,
            has_gt_f.astype(bool), pfh_f, rf, path_exists, rf, rpt, sem)


# speedup vs baseline: 858.2592x; 1.0152x over previous
import dataclasses
import functools

import jax
import jax.numpy as jnp
from jax import lax
from jax.experimental import pallas as pl
from jax.experimental.pallas import tpu as pltpu
from jax.experimental.pallas import tpu_sc as plsc

E = 3200000
G = 512
A = 2048
V = 10000
VP = 10112
PT = 0.5
EPS = 0.05
ALPHA = 1.0
BASE = 10.0
TBONUS = 1.0
ILLEGAL = 1e-08
LP = 0.9
PF1P = 1.0


def _final_math_kernel(stats_ref, hit_ref, ag_ref, ptr_lo_ref, ptr_hi_ref,
                       reached_ref, rs_ref, rf_ref,
                       reward_o, recall_o, success_o, zeros_o, fallback_o,
                       pos_p_o, pos_r_o, pos_f1_o, ans_p_o, ans_r_o, ans_f1_o,
                       path_p_o, path_r_o, path_f1_o, has_gt_o, pfh_o,
                       rpt_o, sem_o):
    selected_total = stats_ref[0:1, :]
    pos_total = stats_ref[1:2, :]
    selected_pos = stats_ref[2:3, :]
    path_total = stats_ref[3:4, :]
    path_hits = stats_ref[4:5, :]
    sel_scores_sum = stats_ref[5:6, :]

    def prf(hits, pred, tgt):
        z = jnp.zeros_like(hits)
        p = jnp.where(pred > 0, hits / jnp.clip(pred, 1.0), z)
        r = jnp.where(tgt > 0, hits / jnp.clip(tgt, 1.0), z)
        f1 = jnp.where(p + r > 0, 2.0 * p * r / jnp.clip(p + r, 1e-08), z)
        return p, r, f1

    fallback = (selected_total == 0).astype(jnp.float32)
    pos_p, pos_r, pos_f1 = prf(selected_pos, selected_total, pos_total)
    label_recall = jnp.where(pos_total > 0,
                             selected_pos / jnp.clip(pos_total, 1.0),
                             jnp.zeros_like(selected_pos))

    onehot = (ag_ref[...] == lax.broadcasted_iota(jnp.int32, (A, G), 1))
    hits = jnp.dot(hit_ref[...].astype(jnp.bfloat16),
                   onehot.astype(jnp.bfloat16),
                   preferred_element_type=jnp.float32)
    ans_counts = (ptr_hi_ref[...] - ptr_lo_ref[...]).astype(jnp.float32)
    reached_total = reached_ref[...]
    ans_p, ans_r, ans_f1 = prf(hits, reached_total, ans_counts)
    has_answers = ans_counts > 0
    recall = jnp.where(has_answers, ans_r, label_recall)

    path_p, path_r, path_f1 = prf(path_hits, selected_total, path_total)
    has_gt_path = path_total > 0
    path_full_hit = jnp.logical_and(has_gt_path, path_hits >= path_total)
    recall = jnp.where(has_gt_path, path_r, recall)
    success = rs_ref[...] > 0.5
    any_path = jnp.any(has_gt_path)
    success = jnp.logical_and(
        success, jnp.logical_or(jnp.logical_not(any_path), path_full_hit))
    rf = rf_ref[...]
    connectivity = jnp.clip(rf + EPS, 1e-06)
    semantic_mean = jnp.clip(
        jnp.where(selected_total > 0,
                  sel_scores_sum / jnp.clip(selected_total, 1.0),
                  jnp.zeros_like(sel_scores_sum)), 1e-08, 1.0)
    semantic_score = semantic_mean ** ALPHA
    reward_path_term = jnp.full_like(selected_total, LP) ** selected_total
    path_term = jnp.ones_like(reward_path_term)
    path_term = jnp.where(jnp.logical_and(any_path, has_gt_path),
                          jnp.clip(path_f1, 0.001) ** PF1P, path_term)
    reward = jnp.where(success,
                       BASE * reward_path_term * path_term * semantic_score
                       * connectivity,
                       jnp.full_like(connectivity, ILLEGAL))
    reward = jnp.where(jnp.logical_and(any_path, path_full_hit),
                       reward * (1.0 + TBONUS), reward)
    reward = jnp.clip(reward, ILLEGAL)

    reward_o[...] = reward
    recall_o[...] = recall
    success_o[...] = success.astype(jnp.float32)
    zeros_o[...] = jnp.zeros_like(recall)
    fallback_o[...] = fallback
    pos_p_o[...] = pos_p
    pos_r_o[...] = pos_r
    pos_f1_o[...] = pos_f1
    ans_p_o[...] = ans_p
    ans_r_o[...] = ans_r
    ans_f1_o[...] = ans_f1
    path_p_o[...] = path_p
    path_r_o[...] = path_r
    path_f1_o[...] = path_f1
    has_gt_o[...] = has_gt_path.astype(jnp.float32)
    pfh_o[...] = path_full_hit.astype(jnp.float32)
    rpt_o[...] = reward_path_term
    sem_o[...] = semantic_score


def _final_math(stats, answer_hit, ans_graph2d, ptr_lo, ptr_hi, reached_total,
                rs_f, rf):
    o = jax.ShapeDtypeStruct((1, G), jnp.float32)
    return pl.pallas_call(
        _final_math_kernel,
        out_shape=[o] * 18,
    )(stats, answer_hit, ans_graph2d, ptr_lo, ptr_hi, reached_total, rs_f, rf)


NC = 2
NS = 16
NW = NC * NS
GPW = G // NW
BLK = 8192
NROW = BLK // 128
T = G * VP


def _sc_compiler_params():
    cp = pltpu.CompilerParams()
    if "needs_layout_passes" in pltpu.CompilerParams.__dataclass_fields__:
        cp = dataclasses.replace(cp, needs_layout_passes=False)
    return cp


def _sc_main(tl, hd, sel_f, lab, sco, pth, start520):
    mesh = plsc.VectorSubcoreMesh(core_axis_name="c", subcore_axis_name="s")

    @functools.partial(
        pl.kernel,
        out_type=[jax.ShapeDtypeStruct((NW, 8, 16), jnp.float32),
                  jax.ShapeDtypeStruct((T,), jnp.float32)],
        mesh=mesh,
        scratch_types=[
            pltpu.VMEM((520,), jnp.int32),
            pltpu.VMEM((BLK,), jnp.int32),
            pltpu.VMEM((BLK,), jnp.int32),
            pltpu.VMEM((BLK,), jnp.float32),
            pltpu.VMEM((BLK,), jnp.float32),
            pltpu.VMEM((BLK,), jnp.float32),
            pltpu.VMEM((BLK,), jnp.float32),
            pltpu.VMEM((VP,), jnp.float32),
            pltpu.VMEM((8, 16), jnp.float32),
            pltpu.SemaphoreType.DMA,
            pltpu.SemaphoreType.DMA,
            pltpu.SemaphoreType.DMA,
        ],
        compiler_params=_sc_compiler_params())
    def k(tl_h, hd_h, sel_h, lab_h, sco_h, pth_h, st_h,
          seg_o, table_o,
          startv, b_tl, b_hd, b_sel, b_lab, b_sco, b_pth, rowbuf, stage,
          sem_in, sem_t, sem_out):
        wid = lax.axis_index("s") * NC + lax.axis_index("c")
        g0 = wid * GPW
        lane = lax.iota(jnp.int32, 16)
        zero16 = jnp.zeros((16,), jnp.float32)
        one16 = jnp.full((16,), 1.0, jnp.float32)

        pltpu.async_copy(st_h, startv, sem_in).wait()

        def graph_body(j, rows):
            g = g0 + j
            iv = jnp.where(lane < 8, g, g + 1)
            sv = plsc.load_gather(startv, [iv])
            es = jnp.max(jnp.where(lane < 8, sv, 0))
            ee = jnp.max(jnp.where(lane >= 8, sv, 0))
            es8 = (es // 8) * 8
            nst = (ee - es8 + BLK - 1) // BLK

            @pl.loop(0, VP // 16)
            def _(i):
                rowbuf[pl.ds(i * 16, 16)] = zero16

            def block(i, sums):
                off = pl.multiple_of(jnp.minimum(es8 + i * BLK, E - BLK), 8)
                cps = [
                    pltpu.async_copy(tl_h.at[pl.ds(off, BLK)], b_tl, sem_in),
                    pltpu.async_copy(hd_h.at[pl.ds(off, BLK)], b_hd, sem_in),
                    pltpu.async_copy(sel_h.at[pl.ds(off, BLK)], b_sel, sem_in),
                    pltpu.async_copy(lab_h.at[pl.ds(off, BLK)], b_lab, sem_in),
                    pltpu.async_copy(sco_h.at[pl.ds(off, BLK)], b_sco, sem_in),
                    pltpu.async_copy(pth_h.at[pl.ds(off, BLK)], b_pth, sem_in),
                ]
                for c in cps:
                    c.wait()

                def row(j2, sums):
                    s0, s1, s2, s3, s4, s5 = sums
                    for m in range(8):
                        base = j2 * 128 + m * 16
                        gi = off + base + lane
                        m_in = jnp.logical_and(gi >= es, gi < ee)
                        selb = b_sel[pl.ds(base, 16)] > 0.0
                        msel = jnp.logical_and(m_in, selb)
                        plsc.store_scatter(rowbuf, [b_tl[pl.ds(base, 16)]],
                                           one16, mask=msel)
                        plsc.store_scatter(rowbuf, [b_hd[pl.ds(base, 16)]],
                                           one16, mask=msel)
                        pos_v = (b_lab[pl.ds(base, 16)] > PT
                                 ).astype(jnp.float32)
                        eff_v = jnp.clip(b_sco[pl.ds(base, 16)], 1e-08, 1.0)
                        pth_v = b_pth[pl.ds(base, 16)]
                        m_in_f = m_in.astype(jnp.float32)
                        msel_f = msel.astype(jnp.float32)
                        s0 = s0 + msel_f
                        s1 = s1 + m_in_f * pos_v
                        s2 = s2 + msel_f * pos_v
                        s3 = s3 + m_in_f * pth_v
                        s4 = s4 + msel_f * pth_v
                        s5 = s5 + msel_f * eff_v
                    return (s0, s1, s2, s3, s4, s5)

                return pl.loop(0, NROW, init_carry=sums)(row)

            sums = lax.fori_loop(0, nst, block, (zero16,) * 6)
            pltpu.async_copy(
                rowbuf,
                table_o.at[pl.ds(pl.multiple_of(g * VP, 8), VP)],
                sem_t).wait()
            return tuple(
                jnp.where(lane == j, jnp.sum(s), r)
                for s, r in zip(sums, rows))

        rows = pl.loop(0, GPW, init_carry=(zero16,) * 6)(graph_body)
        for s in range(6):
            stage[s, :] = rows[s]
        pltpu.async_copy(stage, seg_o.at[wid], sem_out).wait()

    return k(tl, hd, sel_f, lab, sco, pth, start520)


def _sc_gather(table, aidx):
    mesh = plsc.VectorSubcoreMesh(core_axis_name="c", subcore_axis_name="s")
    apw = A // NW

    @functools.partial(
        pl.kernel,
        out_type=jax.ShapeDtypeStruct((A,), jnp.float32),
        mesh=mesh,
        scratch_types=[pltpu.VMEM((apw,), jnp.int32),
                       pltpu.VMEM((apw,), jnp.float32),
                       pltpu.SemaphoreType.DMA],
        compiler_params=_sc_compiler_params())
    def k(tab_h, ai_h, out_h, ai_v, hit_v, sem):
        wid = lax.axis_index("s") * NC + lax.axis_index("c")
        base = wid * apw
        pltpu.async_copy(ai_h.at[pl.ds(base, apw)], ai_v, sem).wait()
        pltpu.async_copy(tab_h.at[ai_v], hit_v, sem).wait()
        pltpu.async_copy(hit_v, out_h.at[pl.ds(base, apw)], sem).wait()

    return k(table, aidx)


def _rowsum_kernel(t_ref, o_ref):
    o_ref[...] = jnp.sum(t_ref[...], axis=1, keepdims=True)


def _table_rowsum(table2d):
    return pl.pallas_call(
        _rowsum_kernel,
        grid=(G // 8,),
        in_specs=[pl.BlockSpec((8, VP), lambda i: (i, 0))],
        out_specs=pl.BlockSpec((8, 1), lambda i: (i, 0)),
        out_shape=jax.ShapeDtypeStruct((G, 1), jnp.float32),
    )(table2d)


def kernel(selected_mask, edge_labels, edge_scores, edge_batch, edge_heads,
           edge_tails, answer_entity_ids, answer_ptr, path_mask, path_exists,
           reach_success, reach_fraction):
    sel_f = selected_mask.astype(jnp.float32)
    path_f = path_mask.astype(jnp.float32)
    eb = edge_batch.astype(jnp.int32)

    start = jnp.searchsorted(eb, jnp.arange(G + 1), side="left"
                             ).astype(jnp.int32)
    start520 = jnp.pad(start, (0, 520 - (G + 1)))

    seg_out, table = _sc_main(edge_tails.astype(jnp.int32),
                              edge_heads.astype(jnp.int32), sel_f,
                              edge_labels, edge_scores, path_f, start520)
    stats = jnp.transpose(seg_out, (1, 0, 2)).reshape(8, G)

    ans_graph = jnp.clip(
        jnp.searchsorted(answer_ptr, jnp.arange(A), side="right") - 1, 0, G - 1
    ).astype(jnp.int32)
    aidx = ans_graph * VP + answer_entity_ids.astype(jnp.int32)
    answer_hit = _sc_gather(table, aidx).reshape(1, A)
    table2d = table.reshape(G, VP)

    reached_total = _table_rowsum(table2d).reshape(1, G)
    outs = _final_math(
        stats,
        answer_hit,
        ans_graph.reshape(A, 1),
        answer_ptr[:-1].reshape(1, G).astype(jnp.int32),
        answer_ptr[1:].reshape(1, G).astype(jnp.int32),
        reached_total,
        reach_success.astype(jnp.float32).reshape(1, G),
        reach_fraction.astype(jnp.float32).reshape(1, G),
    )
    (reward, recall, success_f, zeros, fallback, pos_p, pos_r, pos_f1, ans_p,
     ans_r, ans_f1, path_p, path_r, path_f1, has_gt_f, pfh_f, rpt,
     sem) = [o.reshape(G) for o in outs]
    rf = reach_fraction.astype(jnp.float32)
    return (reward, recall, success_f, zeros, fallback, pos_p, pos_r, pos_f1,
            ans_p, ans_r, ans_f1, path_p, path_r, path_f1,
            has_gt_f.astype(bool), pfh_f, rf, path_exists, rf, rpt, sem)


# speedup vs baseline: 901.6489x; 1.0506x over previous
import dataclasses
import functools

import jax
import jax.numpy as jnp
from jax import lax
from jax.experimental import pallas as pl
from jax.experimental.pallas import tpu as pltpu
from jax.experimental.pallas import tpu_sc as plsc

E = 3200000
G = 512
A = 2048
V = 10000
VP = 10112
PT = 0.5
EPS = 0.05
ALPHA = 1.0
BASE = 10.0
TBONUS = 1.0
ILLEGAL = 1e-08
LP = 0.9
PF1P = 1.0


def _final_math_kernel(stats_ref, hit_ref, ag_ref, ptr_lo_ref, ptr_hi_ref,
                       reached_ref, rs_ref, rf_ref,
                       reward_o, recall_o, success_o, zeros_o, fallback_o,
                       pos_p_o, pos_r_o, pos_f1_o, ans_p_o, ans_r_o, ans_f1_o,
                       path_p_o, path_r_o, path_f1_o, has_gt_o, pfh_o,
                       rpt_o, sem_o):
    selected_total = stats_ref[0:1, :]
    pos_total = stats_ref[1:2, :]
    selected_pos = stats_ref[2:3, :]
    path_total = stats_ref[3:4, :]
    path_hits = stats_ref[4:5, :]
    sel_scores_sum = stats_ref[5:6, :]

    def prf(hits, pred, tgt):
        z = jnp.zeros_like(hits)
        p = jnp.where(pred > 0, hits / jnp.clip(pred, 1.0), z)
        r = jnp.where(tgt > 0, hits / jnp.clip(tgt, 1.0), z)
        f1 = jnp.where(p + r > 0, 2.0 * p * r / jnp.clip(p + r, 1e-08), z)
        return p, r, f1

    fallback = (selected_total == 0).astype(jnp.float32)
    pos_p, pos_r, pos_f1 = prf(selected_pos, selected_total, pos_total)
    label_recall = jnp.where(pos_total > 0,
                             selected_pos / jnp.clip(pos_total, 1.0),
                             jnp.zeros_like(selected_pos))

    onehot = (ag_ref[...] == lax.broadcasted_iota(jnp.int32, (A, G), 1))
    hits = jnp.dot(hit_ref[...].astype(jnp.bfloat16),
                   onehot.astype(jnp.bfloat16),
                   preferred_element_type=jnp.float32)
    ans_counts = (ptr_hi_ref[...] - ptr_lo_ref[...]).astype(jnp.float32)
    reached_total = reached_ref[...]
    ans_p, ans_r, ans_f1 = prf(hits, reached_total, ans_counts)
    has_answers = ans_counts > 0
    recall = jnp.where(has_answers, ans_r, label_recall)

    path_p, path_r, path_f1 = prf(path_hits, selected_total, path_total)
    has_gt_path = path_total > 0
    path_full_hit = jnp.logical_and(has_gt_path, path_hits >= path_total)
    recall = jnp.where(has_gt_path, path_r, recall)
    success = rs_ref[...] > 0.5
    any_path = jnp.any(has_gt_path)
    success = jnp.logical_and(
        success, jnp.logical_or(jnp.logical_not(any_path), path_full_hit))
    rf = rf_ref[...]
    connectivity = jnp.clip(rf + EPS, 1e-06)
    semantic_mean = jnp.clip(
        jnp.where(selected_total > 0,
                  sel_scores_sum / jnp.clip(selected_total, 1.0),
                  jnp.zeros_like(sel_scores_sum)), 1e-08, 1.0)
    semantic_score = semantic_mean ** ALPHA
    reward_path_term = jnp.full_like(selected_total, LP) ** selected_total
    path_term = jnp.ones_like(reward_path_term)
    path_term = jnp.where(jnp.logical_and(any_path, has_gt_path),
                          jnp.clip(path_f1, 0.001) ** PF1P, path_term)
    reward = jnp.where(success,
                       BASE * reward_path_term * path_term * semantic_score
                       * connectivity,
                       jnp.full_like(connectivity, ILLEGAL))
    reward = jnp.where(jnp.logical_and(any_path, path_full_hit),
                       reward * (1.0 + TBONUS), reward)
    reward = jnp.clip(reward, ILLEGAL)

    reward_o[...] = reward
    recall_o[...] = recall
    success_o[...] = success.astype(jnp.float32)
    zeros_o[...] = jnp.zeros_like(recall)
    fallback_o[...] = fallback
    pos_p_o[...] = pos_p
    pos_r_o[...] = pos_r
    pos_f1_o[...] = pos_f1
    ans_p_o[...] = ans_p
    ans_r_o[...] = ans_r
    ans_f1_o[...] = ans_f1
    path_p_o[...] = path_p
    path_r_o[...] = path_r
    path_f1_o[...] = path_f1
    has_gt_o[...] = has_gt_path.astype(jnp.float32)
    pfh_o[...] = path_full_hit.astype(jnp.float32)
    rpt_o[...] = reward_path_term
    sem_o[...] = semantic_score


def _final_math(stats, answer_hit, ans_graph2d, ptr_lo, ptr_hi, reached_total,
                rs_f, rf):
    o = jax.ShapeDtypeStruct((1, G), jnp.float32)
    return pl.pallas_call(
        _final_math_kernel,
        out_shape=[o] * 18,
    )(stats, answer_hit, ans_graph2d, ptr_lo, ptr_hi, reached_total, rs_f, rf)


NC = 2
NS = 16
NW = NC * NS
GPW = G // NW
BLK = 8192
NROW = BLK // 128
T = G * VP


def _sc_compiler_params():
    cp = pltpu.CompilerParams()
    if "needs_layout_passes" in pltpu.CompilerParams.__dataclass_fields__:
        cp = dataclasses.replace(cp, needs_layout_passes=False)
    return cp


def _sc_main(tl, hd, sel_f, lab, sco, pth, start520):
    mesh = plsc.VectorSubcoreMesh(core_axis_name="c", subcore_axis_name="s")

    @functools.partial(
        pl.kernel,
        out_type=[jax.ShapeDtypeStruct((NW, 8, 16), jnp.float32),
                  jax.ShapeDtypeStruct((T,), jnp.float32)],
        mesh=mesh,
        scratch_types=[
            pltpu.VMEM((520,), jnp.int32),
            pltpu.VMEM((BLK,), jnp.int32),
            pltpu.VMEM((BLK,), jnp.int32),
            pltpu.VMEM((BLK,), jnp.float32),
            pltpu.VMEM((BLK,), jnp.float32),
            pltpu.VMEM((BLK,), jnp.float32),
            pltpu.VMEM((BLK,), jnp.float32),
            pltpu.VMEM((VP,), jnp.float32),
            pltpu.VMEM((8, 16), jnp.float32),
            pltpu.SemaphoreType.DMA,
            pltpu.SemaphoreType.DMA,
            pltpu.SemaphoreType.DMA,
        ],
        compiler_params=_sc_compiler_params())
    def k(tl_h, hd_h, sel_h, lab_h, sco_h, pth_h, st_h,
          seg_o, table_o,
          startv, b_tl, b_hd, b_sel, b_lab, b_sco, b_pth, rowbuf, stage,
          sem_in, sem_t, sem_out):
        wid = lax.axis_index("s") * NC + lax.axis_index("c")
        g0 = wid * GPW
        lane = lax.iota(jnp.int32, 16)
        zero16 = jnp.zeros((16,), jnp.float32)
        one16 = jnp.full((16,), 1.0, jnp.float32)

        pltpu.async_copy(st_h, startv, sem_in).wait()

        def graph_body(j, rows):
            g = g0 + j
            iv = jnp.where(lane < 8, g, g + 1)
            sv = plsc.load_gather(startv, [iv])
            es = jnp.max(jnp.where(lane < 8, sv, 0))
            ee = jnp.max(jnp.where(lane >= 8, sv, 0))
            es8 = (es // 8) * 8
            nst = (ee - es8 + BLK - 1) // BLK

            @pl.loop(0, VP // 16)
            def _(i):
                rowbuf[pl.ds(i * 16, 16)] = zero16

            def block(i, sums):
                off = pl.multiple_of(jnp.minimum(es8 + i * BLK, E - BLK), 8)
                cps = [
                    pltpu.async_copy(tl_h.at[pl.ds(off, BLK)], b_tl, sem_in),
                    pltpu.async_copy(hd_h.at[pl.ds(off, BLK)], b_hd, sem_in),
                    pltpu.async_copy(sel_h.at[pl.ds(off, BLK)], b_sel, sem_in),
                    pltpu.async_copy(lab_h.at[pl.ds(off, BLK)], b_lab, sem_in),
                    pltpu.async_copy(sco_h.at[pl.ds(off, BLK)], b_sco, sem_in),
                    pltpu.async_copy(pth_h.at[pl.ds(off, BLK)], b_pth, sem_in),
                ]
                for c in cps:
                    c.wait()

                def row(j2, sums):
                    s0, s1, s2, s3, s4, s5 = sums
                    for m in range(8):
                        base = j2 * 128 + m * 16
                        gi = off + base + lane
                        m_in = jnp.logical_and(gi >= es, gi < ee)
                        selb = b_sel[pl.ds(base, 16)] > 0.0
                        msel = jnp.logical_and(m_in, selb)
                        plsc.store_scatter(rowbuf, [b_tl[pl.ds(base, 16)]],
                                           one16, mask=msel)
                        plsc.store_scatter(rowbuf, [b_hd[pl.ds(base, 16)]],
                                           one16, mask=msel)
                        pos_v = (b_lab[pl.ds(base, 16)] > PT
                                 ).astype(jnp.float32)
                        eff_v = jnp.clip(b_sco[pl.ds(base, 16)], 1e-08, 1.0)
                        pth_v = b_pth[pl.ds(base, 16)]
                        m_in_f = m_in.astype(jnp.float32)
                        msel_f = msel.astype(jnp.float32)
                        s0 = s0 + msel_f
                        s1 = s1 + m_in_f * pos_v
                        s2 = s2 + msel_f * pos_v
                        s3 = s3 + m_in_f * pth_v
                        s4 = s4 + msel_f * pth_v
                        s5 = s5 + msel_f * eff_v
                    return (s0, s1, s2, s3, s4, s5)

                return pl.loop(0, NROW, init_carry=sums)(row)

            sums = lax.fori_loop(0, nst, block, (zero16,) * 6)
            pltpu.async_copy(
                rowbuf,
                table_o.at[pl.ds(pl.multiple_of(g * VP, 8), VP)],
                sem_t).wait()
            return tuple(
                jnp.where(lane == j, jnp.sum(s), r)
                for s, r in zip(sums, rows))

        rows = pl.loop(0, GPW, init_carry=(zero16,) * 6)(graph_body)
        for s in range(6):
            stage[s, :] = rows[s]
        pltpu.async_copy(stage, seg_o.at[wid], sem_out).wait()

    return k(tl, hd, sel_f, lab, sco, pth, start520)


def _sc_gather(table, aidx):
    mesh = plsc.VectorSubcoreMesh(core_axis_name="c", subcore_axis_name="s")
    apw = A // NW

    @functools.partial(
        pl.kernel,
        out_type=jax.ShapeDtypeStruct((A,), jnp.float32),
        mesh=mesh,
        scratch_types=[pltpu.VMEM((apw,), jnp.int32),
                       pltpu.VMEM((apw,), jnp.float32),
                       pltpu.SemaphoreType.DMA],
        compiler_params=_sc_compiler_params())
    def k(tab_h, ai_h, out_h, ai_v, hit_v, sem):
        wid = lax.axis_index("s") * NC + lax.axis_index("c")
        base = wid * apw
        pltpu.async_copy(ai_h.at[pl.ds(base, apw)], ai_v, sem).wait()
        pltpu.async_copy(tab_h.at[ai_v], hit_v, sem).wait()
        pltpu.async_copy(hit_v, out_h.at[pl.ds(base, apw)], sem).wait()

    return k(table, aidx)


def _rowsum_kernel(t_ref, o_ref):
    o_ref[...] = jnp.sum(t_ref[...], axis=1, keepdims=True)


def _table_rowsum(table2d):
    return pl.pallas_call(
        _rowsum_kernel,
        grid=(G // 64,),
        in_specs=[pl.BlockSpec((64, VP), lambda i: (i, 0))],
        out_specs=pl.BlockSpec((64, 1), lambda i: (i, 0)),
        out_shape=jax.ShapeDtypeStruct((G, 1), jnp.float32),
    )(table2d)


def kernel(selected_mask, edge_labels, edge_scores, edge_batch, edge_heads,
           edge_tails, answer_entity_ids, answer_ptr, path_mask, path_exists,
           reach_success, reach_fraction):
    sel_f = selected_mask.astype(jnp.float32)
    path_f = path_mask.astype(jnp.float32)
    eb = edge_batch.astype(jnp.int32)

    start = jnp.searchsorted(eb, jnp.arange(G + 1), side="left"
                             ).astype(jnp.int32)
    start520 = jnp.pad(start, (0, 520 - (G + 1)))

    seg_out, table = _sc_main(edge_tails.astype(jnp.int32),
                              edge_heads.astype(jnp.int32), sel_f,
                              edge_labels, edge_scores, path_f, start520)
    stats = jnp.transpose(seg_out, (1, 0, 2)).reshape(8, G)

    ans_graph = jnp.clip(
        jnp.searchsorted(answer_ptr, jnp.arange(A), side="right") - 1, 0, G - 1
    ).astype(jnp.int32)
    aidx = ans_graph * VP + answer_entity_ids.astype(jnp.int32)
    answer_hit = _sc_gather(table, aidx).reshape(1, A)
    table2d = table.reshape(G, VP)

    reached_total = _table_rowsum(table2d).reshape(1, G)
    outs = _final_math(
        stats,
        answer_hit,
        ans_graph.reshape(A, 1),
        answer_ptr[:-1].reshape(1, G).astype(jnp.int32),
        answer_ptr[1:].reshape(1, G).astype(jnp.int32),
        reached_total,
        reach_success.astype(jnp.float32).reshape(1, G),
        reach_fraction.astype(jnp.float32).reshape(1, G),
    )
    (reward, recall, success_f, zeros, fallback, pos_p, pos_r, pos_f1, ans_p,
     ans_r, ans_f1, path_p, path_r, path_f1, has_gt_f, pfh_f, rpt,
     sem) = [o.reshape(G) for o in outs]
    rf = reach_fraction.astype(jnp.float32)
    return (reward, recall, success_f, zeros, fallback, pos_p, pos_r, pos_f1,
            ans_p, ans_r, ans_f1, path_p, path_r, path_f1,
            has_gt_f.astype(bool), pfh_f, rf, path_exists, rf, rpt, sem)


# speedup vs baseline: 903.7340x; 1.0023x over previous
import dataclasses
import functools

import jax
import jax.numpy as jnp
from jax import lax
from jax.experimental import pallas as pl
from jax.experimental.pallas import tpu as pltpu
from jax.experimental.pallas import tpu_sc as plsc

E = 3200000
G = 512
A = 2048
V = 10000
VP = 10112
PT = 0.5
EPS = 0.05
ALPHA = 1.0
BASE = 10.0
TBONUS = 1.0
ILLEGAL = 1e-08
LP = 0.9
PF1P = 1.0


def _final_math_kernel(stats_ref, hit_ref, ag_ref, ptr_lo_ref, ptr_hi_ref,
                       reached_ref, rs_ref, rf_ref,
                       reward_o, recall_o, success_o, zeros_o, fallback_o,
                       pos_p_o, pos_r_o, pos_f1_o, ans_p_o, ans_r_o, ans_f1_o,
                       path_p_o, path_r_o, path_f1_o, has_gt_o, pfh_o,
                       rpt_o, sem_o):
    selected_total = stats_ref[0:1, :]
    pos_total = stats_ref[1:2, :]
    selected_pos = stats_ref[2:3, :]
    path_total = stats_ref[3:4, :]
    path_hits = stats_ref[4:5, :]
    sel_scores_sum = stats_ref[5:6, :]

    def prf(hits, pred, tgt):
        z = jnp.zeros_like(hits)
        p = jnp.where(pred > 0, hits / jnp.clip(pred, 1.0), z)
        r = jnp.where(tgt > 0, hits / jnp.clip(tgt, 1.0), z)
        f1 = jnp.where(p + r > 0, 2.0 * p * r / jnp.clip(p + r, 1e-08), z)
        return p, r, f1

    fallback = (selected_total == 0).astype(jnp.float32)
    pos_p, pos_r, pos_f1 = prf(selected_pos, selected_total, pos_total)
    label_recall = jnp.where(pos_total > 0,
                             selected_pos / jnp.clip(pos_total, 1.0),
                             jnp.zeros_like(selected_pos))

    onehot = (ag_ref[...] == lax.broadcasted_iota(jnp.int32, (A, G), 1))
    hits = jnp.dot(hit_ref[...].astype(jnp.bfloat16),
                   onehot.astype(jnp.bfloat16),
                   preferred_element_type=jnp.float32)
    ans_counts = (ptr_hi_ref[...] - ptr_lo_ref[...]).astype(jnp.float32)
    reached_total = reached_ref[...]
    ans_p, ans_r, ans_f1 = prf(hits, reached_total, ans_counts)
    has_answers = ans_counts > 0
    recall = jnp.where(has_answers, ans_r, label_recall)

    path_p, path_r, path_f1 = prf(path_hits, selected_total, path_total)
    has_gt_path = path_total > 0
    path_full_hit = jnp.logical_and(has_gt_path, path_hits >= path_total)
    recall = jnp.where(has_gt_path, path_r, recall)
    success = rs_ref[...] > 0.5
    any_path = jnp.any(has_gt_path)
    success = jnp.logical_and(
        success, jnp.logical_or(jnp.logical_not(any_path), path_full_hit))
    rf = rf_ref[...]
    connectivity = jnp.clip(rf + EPS, 1e-06)
    semantic_mean = jnp.clip(
        jnp.where(selected_total > 0,
                  sel_scores_sum / jnp.clip(selected_total, 1.0),
                  jnp.zeros_like(sel_scores_sum)), 1e-08, 1.0)
    semantic_score = semantic_mean ** ALPHA
    reward_path_term = jnp.full_like(selected_total, LP) ** selected_total
    path_term = jnp.ones_like(reward_path_term)
    path_term = jnp.where(jnp.logical_and(any_path, has_gt_path),
                          jnp.clip(path_f1, 0.001) ** PF1P, path_term)
    reward = jnp.where(success,
                       BASE * reward_path_term * path_term * semantic_score
                       * connectivity,
                       jnp.full_like(connectivity, ILLEGAL))
    reward = jnp.where(jnp.logical_and(any_path, path_full_hit),
                       reward * (1.0 + TBONUS), reward)
    reward = jnp.clip(reward, ILLEGAL)

    reward_o[...] = reward
    recall_o[...] = recall
    success_o[...] = success.astype(jnp.float32)
    zeros_o[...] = jnp.zeros_like(recall)
    fallback_o[...] = fallback
    pos_p_o[...] = pos_p
    pos_r_o[...] = pos_r
    pos_f1_o[...] = pos_f1
    ans_p_o[...] = ans_p
    ans_r_o[...] = ans_r
    ans_f1_o[...] = ans_f1
    path_p_o[...] = path_p
    path_r_o[...] = path_r
    path_f1_o[...] = path_f1
    has_gt_o[...] = has_gt_path.astype(jnp.float32)
    pfh_o[...] = path_full_hit.astype(jnp.float32)
    rpt_o[...] = reward_path_term
    sem_o[...] = semantic_score


def _final_math(stats, answer_hit, ans_graph2d, ptr_lo, ptr_hi, reached_total,
                rs_f, rf):
    o = jax.ShapeDtypeStruct((1, G), jnp.float32)
    return pl.pallas_call(
        _final_math_kernel,
        out_shape=[o] * 18,
    )(stats, answer_hit, ans_graph2d, ptr_lo, ptr_hi, reached_total, rs_f, rf)


NC = 2
NS = 16
NW = NC * NS
GPW = G // NW
BLK = 8192
NROW = BLK // 128
T = G * VP


def _sc_compiler_params():
    cp = pltpu.CompilerParams()
    if "needs_layout_passes" in pltpu.CompilerParams.__dataclass_fields__:
        cp = dataclasses.replace(cp, needs_layout_passes=False)
    return cp


def _sc_main(tl, hd, sel_f, lab, sco, pth, start520):
    mesh = plsc.VectorSubcoreMesh(core_axis_name="c", subcore_axis_name="s")

    @functools.partial(
        pl.kernel,
        out_type=[jax.ShapeDtypeStruct((NW, 8, 16), jnp.float32),
                  jax.ShapeDtypeStruct((T,), jnp.float32)],
        mesh=mesh,
        scratch_types=[
            pltpu.VMEM((520,), jnp.int32),
            pltpu.VMEM((BLK,), jnp.int32),
            pltpu.VMEM((BLK,), jnp.int32),
            pltpu.VMEM((BLK,), jnp.float32),
            pltpu.VMEM((BLK,), jnp.float32),
            pltpu.VMEM((BLK,), jnp.float32),
            pltpu.VMEM((BLK,), jnp.float32),
            pltpu.VMEM((VP,), jnp.float32),
            pltpu.VMEM((8, 16), jnp.float32),
            pltpu.SemaphoreType.DMA,
            pltpu.SemaphoreType.DMA,
            pltpu.SemaphoreType.DMA,
        ],
        compiler_params=_sc_compiler_params())
    def k(tl_h, hd_h, sel_h, lab_h, sco_h, pth_h, st_h,
          seg_o, table_o,
          startv, b_tl, b_hd, b_sel, b_lab, b_sco, b_pth, rowbuf, stage,
          sem_in, sem_t, sem_out):
        wid = lax.axis_index("s") * NC + lax.axis_index("c")
        g0 = wid * GPW
        lane = lax.iota(jnp.int32, 16)
        zero16 = jnp.zeros((16,), jnp.float32)
        one16 = jnp.full((16,), 1.0, jnp.float32)

        pltpu.async_copy(st_h, startv, sem_in).wait()

        def graph_body(j, rows):
            g = g0 + j
            iv = jnp.where(lane < 8, g, g + 1)
            sv = plsc.load_gather(startv, [iv])
            es = jnp.max(jnp.where(lane < 8, sv, 0))
            ee = jnp.max(jnp.where(lane >= 8, sv, 0))
            es8 = (es // 8) * 8
            nst = (ee - es8 + BLK - 1) // BLK

            @pl.loop(0, VP // 16)
            def _(i):
                rowbuf[pl.ds(i * 16, 16)] = zero16

            def block(i, sums):
                off = pl.multiple_of(jnp.minimum(es8 + i * BLK, E - BLK), 8)
                cps = [
                    pltpu.async_copy(tl_h.at[pl.ds(off, BLK)], b_tl, sem_in),
                    pltpu.async_copy(hd_h.at[pl.ds(off, BLK)], b_hd, sem_in),
                    pltpu.async_copy(sel_h.at[pl.ds(off, BLK)], b_sel, sem_in),
                    pltpu.async_copy(lab_h.at[pl.ds(off, BLK)], b_lab, sem_in),
                    pltpu.async_copy(sco_h.at[pl.ds(off, BLK)], b_sco, sem_in),
                    pltpu.async_copy(pth_h.at[pl.ds(off, BLK)], b_pth, sem_in),
                ]
                for c in cps:
                    c.wait()

                def row(j2, sums):
                    s0, s1, s2, s3, s4, s5 = sums
                    for m in range(8):
                        base = j2 * 128 + m * 16
                        gi = off + base + lane
                        m_in = jnp.logical_and(gi >= es, gi < ee)
                        selb = b_sel[pl.ds(base, 16)] > 0.0
                        msel = jnp.logical_and(m_in, selb)
                        plsc.store_scatter(rowbuf, [b_tl[pl.ds(base, 16)]],
                                           one16, mask=msel)
                        plsc.store_scatter(rowbuf, [b_hd[pl.ds(base, 16)]],
                                           one16, mask=msel)
                        pos_v = (b_lab[pl.ds(base, 16)] > PT
                                 ).astype(jnp.float32)
                        eff_v = jnp.clip(b_sco[pl.ds(base, 16)], 1e-08, 1.0)
                        pth_v = b_pth[pl.ds(base, 16)]
                        m_in_f = m_in.astype(jnp.float32)
                        msel_f = msel.astype(jnp.float32)
                        s0 = s0 + msel_f
                        s1 = s1 + m_in_f * pos_v
                        s2 = s2 + msel_f * pos_v
                        s3 = s3 + m_in_f * pth_v
                        s4 = s4 + msel_f * pth_v
                        s5 = s5 + msel_f * eff_v
                    return (s0, s1, s2, s3, s4, s5)

                return pl.loop(0, NROW, init_carry=sums, unroll=2)(row)

            sums = lax.fori_loop(0, nst, block, (zero16,) * 6)
            pltpu.async_copy(
                rowbuf,
                table_o.at[pl.ds(pl.multiple_of(g * VP, 8), VP)],
                sem_t).wait()
            return tuple(
                jnp.where(lane == j, jnp.sum(s), r)
                for s, r in zip(sums, rows))

        rows = pl.loop(0, GPW, init_carry=(zero16,) * 6)(graph_body)
        for s in range(6):
            stage[s, :] = rows[s]
        pltpu.async_copy(stage, seg_o.at[wid], sem_out).wait()

    return k(tl, hd, sel_f, lab, sco, pth, start520)


def _sc_gather(table, aidx):
    mesh = plsc.VectorSubcoreMesh(core_axis_name="c", subcore_axis_name="s")
    apw = A // NW

    @functools.partial(
        pl.kernel,
        out_type=jax.ShapeDtypeStruct((A,), jnp.float32),
        mesh=mesh,
        scratch_types=[pltpu.VMEM((apw,), jnp.int32),
                       pltpu.VMEM((apw,), jnp.float32),
                       pltpu.SemaphoreType.DMA],
        compiler_params=_sc_compiler_params())
    def k(tab_h, ai_h, out_h, ai_v, hit_v, sem):
        wid = lax.axis_index("s") * NC + lax.axis_index("c")
        base = wid * apw
        pltpu.async_copy(ai_h.at[pl.ds(base, apw)], ai_v, sem).wait()
        pltpu.async_copy(tab_h.at[ai_v], hit_v, sem).wait()
        pltpu.async_copy(hit_v, out_h.at[pl.ds(base, apw)], sem).wait()

    return k(table, aidx)


def _rowsum_kernel(t_ref, o_ref):
    o_ref[...] = jnp.sum(t_ref[...], axis=1, keepdims=True)


def _table_rowsum(table2d):
    return pl.pallas_call(
        _rowsum_kernel,
        grid=(G // 64,),
        in_specs=[pl.BlockSpec((64, VP), lambda i: (i, 0))],
        out_specs=pl.BlockSpec((64, 1), lambda i: (i, 0)),
        out_shape=jax.ShapeDtypeStruct((G, 1), jnp.float32),
    )(table2d)


def kernel(selected_mask, edge_labels, edge_scores, edge_batch, edge_heads,
           edge_tails, answer_entity_ids, answer_ptr, path_mask, path_exists,
           reach_success, reach_fraction):
    sel_f = selected_mask.astype(jnp.float32)
    path_f = path_mask.astype(jnp.float32)
    eb = edge_batch.astype(jnp.int32)

    start = jnp.searchsorted(eb, jnp.arange(G + 1), side="left"
                             ).astype(jnp.int32)
    start520 = jnp.pad(start, (0, 520 - (G + 1)))

    seg_out, table = _sc_main(edge_tails.astype(jnp.int32),
                              edge_heads.astype(jnp.int32), sel_f,
                              edge_labels, edge_scores, path_f, start520)
    stats = jnp.transpose(seg_out, (1, 0, 2)).reshape(8, G)

    ans_graph = jnp.clip(
        jnp.searchsorted(answer_ptr, jnp.arange(A), side="right") - 1, 0, G - 1
    ).astype(jnp.int32)
    aidx = ans_graph * VP + answer_entity_ids.astype(jnp.int32)
    answer_hit = _sc_gather(table, aidx).reshape(1, A)
    table2d = table.reshape(G, VP)

    reached_total = _table_rowsum(table2d).reshape(1, G)
    outs = _final_math(
        stats,
        answer_hit,
        ans_graph.reshape(A, 1),
        answer_ptr[:-1].reshape(1, G).astype(jnp.int32),
        answer_ptr[1:].reshape(1, G).astype(jnp.int32),
        reached_total,
        reach_success.astype(jnp.float32).reshape(1, G),
        reach_fraction.astype(jnp.float32).reshape(1, G),
    )
    (reward, recall, success_f, zeros, fallback, pos_p, pos_r, pos_f1, ans_p,
     ans_r, ans_f1, path_p, path_r, path_f1, has_gt_f, pfh_f, rpt,
     sem) = [o.reshape(G) for o in outs]
    rf = reach_fraction.astype(jnp.float32)
    return (reward, recall, success_f, zeros, fallback, pos_p, pos_r, pos_f1,
            ans_p, ans_r, ans_f1, path_p, path_r, path_f1,
            has_gt_f.astype(bool), pfh_f, rf, path_exists, rf, rpt, sem)
